# baseline (speedup 1.0000x reference)
import functools
import jax
import jax.numpy as jnp
from jax import lax
from jax.experimental import pallas as pl
from jax.experimental.pallas import tpu as pltpu
from jax.experimental.pallas import tpu_sc as plsc

B, D, E, K = 4096, 768, 8, 2
P = B * K
TR = 256
NP = P + E * TR
NT = NP // TR

_NC, _NS = 2, 16
_NW = _NC * _NS
_GCH = 64
_CCH = 32


def _routing_kernel(x_ref, wg_ref, bg_ref, idx_ref, wts_ref):
    x = x_ref[...]
    glog = jnp.dot(x, wg_ref[...], preferred_element_type=jnp.float32) + bg_ref[...]
    ii = jax.lax.broadcasted_iota(jnp.int32, glog.shape, 1)
    ne = glog.shape[1]
    m1 = jnp.max(glog, axis=1, keepdims=True)
    i1 = jnp.min(jnp.where(glog >= m1, ii, ne), axis=1, keepdims=True)
    neg = jnp.finfo(jnp.float32).min
    g2 = jnp.where(ii == i1, neg, glog)
    m2 = jnp.max(g2, axis=1, keepdims=True)
    i2 = jnp.min(jnp.where(g2 >= m2, ii, ne), axis=1, keepdims=True)
    p2 = jnp.exp(m2 - m1)
    denom = 1.0 + p2
    idx_ref[...] = jnp.concatenate([i1, i2], axis=1)
    wts_ref[...] = jnp.concatenate([1.0 / denom, p2 / denom], axis=1)


def _routing(x, Wg, bg):
    return pl.pallas_call(
        _routing_kernel,
        grid=(1,),
        in_specs=[
            pl.BlockSpec((B, D), lambda i: (0, 0)),
            pl.BlockSpec((D, E), lambda i: (0, 0)),
            pl.BlockSpec((1, E), lambda i: (0, 0)),
        ],
        out_specs=[
            pl.BlockSpec((B, K), lambda i: (0, 0)),
            pl.BlockSpec((B, K), lambda i: (0, 0)),
        ],
        out_shape=[
            jax.ShapeDtypeStruct((B, K), jnp.int32),
            jax.ShapeDtypeStruct((B, K), jnp.float32),
        ],
    )(x, Wg, bg.reshape(1, E))


def _metadata(idx, wts):
    eid = idx.reshape(P)
    wt = wts.reshape(P)
    tok = jnp.repeat(jnp.arange(B, dtype=jnp.int32), K)
    onehot = (eid[:, None] == jnp.arange(E, dtype=jnp.int32)[None, :]).astype(jnp.int32)
    ranks = jnp.cumsum(onehot, axis=0) - onehot
    rank = jnp.take_along_axis(ranks, eid[:, None], axis=1)[:, 0]
    counts = jnp.sum(onehot, axis=0)
    padded = ((counts + TR - 1) // TR) * TR
    poff = jnp.concatenate([jnp.zeros((1,), jnp.int32),
                            jnp.cumsum(padded)[:-1].astype(jnp.int32)])
    dest = poff[eid] + rank
    tok_padded = jnp.zeros((NP,), jnp.int32).at[dest].set(tok)
    wt_padded = jnp.zeros((NP,), jnp.float32).at[dest].set(wt)
    pend = poff + padded
    t_starts = jnp.arange(NT, dtype=jnp.int32) * TR
    tile_expert = jnp.clip(jnp.sum((t_starts[:, None] >= pend[None, :]).astype(jnp.int32),
                                    axis=1), 0, E - 1).astype(jnp.int32)
    dest_even = dest[0::2]
    dest_odd = dest[1::2]
    return dest_even, dest_odd, tok_padded, wt_padded, tile_expert


def _sc_gather_body(x_hbm, tokp_hbm, xg_hbm, idx_v, rows_v, sem):
    wid = lax.axis_index("s") * _NC + lax.axis_index("c")
    rows_per_w = NP // _NW
    base = wid * rows_per_w
    for i in range(rows_per_w // _GCH):
        off = base + i * _GCH
        pltpu.sync_copy(tokp_hbm.at[pl.ds(off, _GCH)], idx_v)
        pltpu.async_copy(x_hbm.at[idx_v], rows_v, sem).wait()
        pltpu.sync_copy(rows_v, xg_hbm.at[pl.ds(off, _GCH)])


def _sc_gather(x, tok_padded):
    k = functools.partial(
        pl.kernel,
        mesh=plsc.VectorSubcoreMesh(core_axis_name="c", subcore_axis_name="s"),
        out_type=jax.ShapeDtypeStruct((NP, D), jnp.float32),
        scratch_types=[
            pltpu.VMEM((_GCH,), jnp.int32),
            pltpu.VMEM((_GCH, D), jnp.float32),
            pltpu.SemaphoreType.DMA,
        ],
    )(_sc_gather_body)
    return k(x, tok_padded)


def _gemm_kernel(te_ref, xg_ref, w1_ref, b1_ref, w2_ref, b2_ref, wt_ref, y_ref):
    xg = xg_ref[...]
    h = jnp.maximum(
        jnp.dot(xg, w1_ref[0], preferred_element_type=jnp.float32) + b1_ref[0], 0.0)
    y = jnp.dot(h, w2_ref[0], preferred_element_type=jnp.float32) + b2_ref[0]
    y_ref[...] = y * wt_ref[...]


def _grouped_gemm(xg, W1, b1, W2, b2, wt_padded, tile_expert):
    grid_spec = pltpu.PrefetchScalarGridSpec(
        num_scalar_prefetch=1,
        grid=(NT,),
        in_specs=[
            pl.BlockSpec((TR, D), lambda t, te: (t, 0)),
            pl.BlockSpec((1, D, D), lambda t, te: (te[t], 0, 0)),
            pl.BlockSpec((1, 1, D), lambda t, te: (te[t], 0, 0)),
            pl.BlockSpec((1, D, D), lambda t, te: (te[t], 0, 0)),
            pl.BlockSpec((1, 1, D), lambda t, te: (te[t], 0, 0)),
            pl.BlockSpec((TR, 1), lambda t, te: (t, 0)),
        ],
        out_specs=pl.BlockSpec((TR, D), lambda t, te: (t, 0)),
    )
    return pl.pallas_call(
        _gemm_kernel,
        grid_spec=grid_spec,
        out_shape=jax.ShapeDtypeStruct((NP, D), jnp.float32),
    )(tile_expert, xg, W1, b1.reshape(E, 1, D), W2, b2.reshape(E, 1, D),
      wt_padded.reshape(NP, 1))


def _sc_combine_body(y_hbm, de_hbm, do_hbm, out_hbm, idx_v, a_v, b_v, o_v, sem):
    wid = lax.axis_index("s") * _NC + lax.axis_index("c")
    tpw = B // _NW
    for i in range(tpw // _CCH):
        off = wid * tpw + i * _CCH
        pltpu.sync_copy(de_hbm.at[pl.ds(off, _CCH)], idx_v)
        pltpu.async_copy(y_hbm.at[idx_v], a_v, sem).wait()
        pltpu.sync_copy(do_hbm.at[pl.ds(off, _CCH)], idx_v)
        pltpu.async_copy(y_hbm.at[idx_v], b_v, sem).wait()

        def body(t, carry):
            for j in range(D // 16):
                sl = pl.ds(j * 16, 16)
                o_v[t, sl] = a_v[t, sl] + b_v[t, sl]
            return carry

        lax.fori_loop(0, _CCH, body, 0)
        pltpu.sync_copy(o_v, out_hbm.at[pl.ds(off, _CCH)])


def _sc_combine(y, dest_even, dest_odd):
    k = functools.partial(
        pl.kernel,
        mesh=plsc.VectorSubcoreMesh(core_axis_name="c", subcore_axis_name="s"),
        out_type=jax.ShapeDtypeStruct((B, D), jnp.float32),
        scratch_types=[
            pltpu.VMEM((_CCH,), jnp.int32),
            pltpu.VMEM((_CCH, D), jnp.float32),
            pltpu.VMEM((_CCH, D), jnp.float32),
            pltpu.VMEM((_CCH, D), jnp.float32),
            pltpu.SemaphoreType.DMA,
        ],
    )(_sc_combine_body)
    return k(y, dest_even, dest_odd)


def kernel(x, Wg, bg, W1, b1, W2, b2):
    idx, wts = _routing(x, Wg, bg)
    dest_even, dest_odd, tok_padded, wt_padded, tile_expert = _metadata(idx, wts)
    xg = _sc_gather(x, tok_padded)
    y = _grouped_gemm(xg, W1, b1, W2, b2, wt_padded, tile_expert)
    return _sc_combine(y, dest_even, dest_odd)


# speedup vs baseline: 1.0685x; 1.0685x over previous
import functools
import jax
import jax.numpy as jnp
from jax import lax
from jax.experimental import pallas as pl
from jax.experimental.pallas import tpu as pltpu
from jax.experimental.pallas import tpu_sc as plsc

B, D, E, K = 4096, 768, 8, 2
P = B * K
TR = 256
NP = P + E * TR
NT = NP // TR

_NC, _NS = 2, 16
_NW = _NC * _NS
_GCH = 64
_CCH = 32


def _routing_kernel(x_ref, wg_ref, bg_ref, idx_ref, wts_ref, dest_ref, pend_ref):
    x = x_ref[...]
    glog = jnp.dot(x, wg_ref[...], preferred_element_type=jnp.float32) + bg_ref[...]
    ii = jax.lax.broadcasted_iota(jnp.int32, glog.shape, 1)
    ne = glog.shape[1]
    m1 = jnp.max(glog, axis=1, keepdims=True)
    i1 = jnp.min(jnp.where(glog >= m1, ii, ne), axis=1, keepdims=True)
    neg = jnp.finfo(jnp.float32).min
    g2 = jnp.where(ii == i1, neg, glog)
    m2 = jnp.max(g2, axis=1, keepdims=True)
    i2 = jnp.min(jnp.where(g2 >= m2, ii, ne), axis=1, keepdims=True)
    p2 = jnp.exp(m2 - m1)
    denom = 1.0 + p2
    idx_ref[...] = jnp.concatenate([i1, i2], axis=1)
    wts_ref[...] = jnp.concatenate([1.0 / denom, p2 / denom], axis=1)

    one = ((ii == i1) | (ii == i2)).astype(jnp.int32)
    inc = one
    sh = 1
    while sh < one.shape[0]:
        z = jnp.zeros((sh, one.shape[1]), jnp.int32)
        inc = inc + jnp.concatenate([z, inc[:-sh, :]], axis=0)
        sh *= 2
    excl = inc - one
    counts = inc[-1:, :]
    padded = ((counts + TR - 1) // TR) * TR
    pinc = padded
    sh = 1
    while sh < ne:
        z = jnp.zeros((1, sh), jnp.int32)
        pinc = pinc + jnp.concatenate([z, pinc[:, :-sh]], axis=1)
        sh *= 2
    poff = pinc - padded
    d1 = jnp.sum(jnp.where(ii == i1, poff + excl, 0), axis=1, keepdims=True)
    d2 = jnp.sum(jnp.where(ii == i2, poff + excl, 0), axis=1, keepdims=True)
    dest_ref[...] = jnp.concatenate([d1, d2], axis=1)
    pend_ref[...] = pinc


def _routing(x, Wg, bg):
    return pl.pallas_call(
        _routing_kernel,
        grid=(1,),
        in_specs=[
            pl.BlockSpec((B, D), lambda i: (0, 0)),
            pl.BlockSpec((D, E), lambda i: (0, 0)),
            pl.BlockSpec((1, E), lambda i: (0, 0)),
        ],
        out_specs=[
            pl.BlockSpec((B, K), lambda i: (0, 0)),
            pl.BlockSpec((B, K), lambda i: (0, 0)),
            pl.BlockSpec((B, K), lambda i: (0, 0)),
            pl.BlockSpec((1, E), lambda i: (0, 0)),
        ],
        out_shape=[
            jax.ShapeDtypeStruct((B, K), jnp.int32),
            jax.ShapeDtypeStruct((B, K), jnp.float32),
            jax.ShapeDtypeStruct((B, K), jnp.int32),
            jax.ShapeDtypeStruct((1, E), jnp.int32),
        ],
    )(x, Wg, bg.reshape(1, E))


def _metadata(idx, wts, dest, pend):
    wt = wts.reshape(P)
    tok = jnp.repeat(jnp.arange(B, dtype=jnp.int32), K)
    destf = dest.reshape(P)
    tok_padded = jnp.zeros((NP,), jnp.int32).at[destf].set(tok)
    wt_padded = jnp.zeros((NP,), jnp.float32).at[destf].set(wt)
    t_starts = jnp.arange(NT, dtype=jnp.int32) * TR
    tile_expert = jnp.clip(jnp.sum((t_starts[:, None] >= pend[0][None, :]).astype(jnp.int32),
                                    axis=1), 0, E - 1).astype(jnp.int32)
    return dest[:, 0], dest[:, 1], tok_padded, wt_padded, tile_expert


def _sc_gather_body(x_hbm, tokp_hbm, xg_hbm, idx_v, rows_a, rows_b,
                    gsem_a, gsem_b, wsem_a, wsem_b):
    wid = lax.axis_index("s") * _NC + lax.axis_index("c")
    rows_per_w = NP // _NW
    n_chunks = rows_per_w // _GCH
    base = wid * rows_per_w
    bufs = (rows_a, rows_b)
    gsems = (gsem_a, gsem_b)
    wsems = (wsem_a, wsem_b)
    pltpu.sync_copy(tokp_hbm.at[pl.ds(base, rows_per_w)], idx_v)
    gh = [None, None]
    wh = [None, None]
    for i in range(n_chunks):
        b = i % 2
        if wh[b] is not None:
            wh[b].wait()
        gh[b] = pltpu.async_copy(
            x_hbm.at[idx_v.at[pl.ds(i * _GCH, _GCH)]], bufs[b], gsems[b])
        if i >= 1:
            pb = (i - 1) % 2
            gh[pb].wait()
            wh[pb] = pltpu.async_copy(
                bufs[pb], xg_hbm.at[pl.ds(base + (i - 1) * _GCH, _GCH)], wsems[pb])
    lb = (n_chunks - 1) % 2
    gh[lb].wait()
    wh[lb] = pltpu.async_copy(
        bufs[lb], xg_hbm.at[pl.ds(base + (n_chunks - 1) * _GCH, _GCH)], wsems[lb])
    for b in range(2):
        if wh[b] is not None:
            wh[b].wait()


def _sc_gather(x, tok_padded):
    k = functools.partial(
        pl.kernel,
        mesh=plsc.VectorSubcoreMesh(core_axis_name="c", subcore_axis_name="s"),
        out_type=jax.ShapeDtypeStruct((NP, D), jnp.float32),
        scratch_types=[
            pltpu.VMEM((NP // _NW,), jnp.int32),
            pltpu.VMEM((_GCH, D), jnp.float32),
            pltpu.VMEM((_GCH, D), jnp.float32),
            pltpu.SemaphoreType.DMA,
            pltpu.SemaphoreType.DMA,
            pltpu.SemaphoreType.DMA,
            pltpu.SemaphoreType.DMA,
        ],
    )(_sc_gather_body)
    return k(x, tok_padded)


def _gemm_kernel(te_ref, xg_ref, w1_ref, b1_ref, w2_ref, b2_ref, wt_ref, y_ref):
    xg = xg_ref[...]
    h = jnp.maximum(
        jnp.dot(xg, w1_ref[0], preferred_element_type=jnp.float32) + b1_ref[0], 0.0)
    y = jnp.dot(h, w2_ref[0], preferred_element_type=jnp.float32) + b2_ref[0]
    y_ref[...] = y * wt_ref[...]


def _grouped_gemm(xg, W1, b1, W2, b2, wt_padded, tile_expert):
    grid_spec = pltpu.PrefetchScalarGridSpec(
        num_scalar_prefetch=1,
        grid=(NT,),
        in_specs=[
            pl.BlockSpec((TR, D), lambda t, te: (t, 0)),
            pl.BlockSpec((1, D, D), lambda t, te: (te[t], 0, 0)),
            pl.BlockSpec((1, 1, D), lambda t, te: (te[t], 0, 0)),
            pl.BlockSpec((1, D, D), lambda t, te: (te[t], 0, 0)),
            pl.BlockSpec((1, 1, D), lambda t, te: (te[t], 0, 0)),
            pl.BlockSpec((TR, 1), lambda t, te: (t, 0)),
        ],
        out_specs=pl.BlockSpec((TR, D), lambda t, te: (t, 0)),
    )
    return pl.pallas_call(
        _gemm_kernel,
        grid_spec=grid_spec,
        out_shape=jax.ShapeDtypeStruct((NP, D), jnp.float32),
    )(tile_expert, xg, W1, b1.reshape(E, 1, D), W2, b2.reshape(E, 1, D),
      wt_padded.reshape(NP, 1))


def _sc_combine_body(y_hbm, de_hbm, do_hbm, out_hbm, de_v, do_v,
                     a0, a1, b0, b1, sa0, sa1, sb0, sb1, w0, w1):
    wid = lax.axis_index("s") * _NC + lax.axis_index("c")
    tpw = B // _NW
    n_chunks = tpw // _CCH
    base = wid * tpw
    a = (a0, a1)
    b = (b0, b1)
    gsa = (sa0, sa1)
    gsb = (sb0, sb1)
    ws = (w0, w1)
    pltpu.sync_copy(de_hbm.at[pl.ds(base, tpw)], de_v)
    pltpu.sync_copy(do_hbm.at[pl.ds(base, tpw)], do_v)
    ga = [None, None]
    gb = [None, None]
    wh = [None, None]

    def _add_pairs(s):
        def body(t, carry):
            for j in range(D // 16):
                sl = pl.ds(j * 16, 16)
                a[s][t, sl] = a[s][t, sl] + b[s][t, sl]
            return carry
        lax.fori_loop(0, _CCH, body, 0)

    for i in range(n_chunks):
        s = i % 2
        if wh[s] is not None:
            wh[s].wait()
        ga[s] = pltpu.async_copy(
            y_hbm.at[de_v.at[pl.ds(i * _CCH, _CCH)]], a[s], gsa[s])
        gb[s] = pltpu.async_copy(
            y_hbm.at[do_v.at[pl.ds(i * _CCH, _CCH)]], b[s], gsb[s])
        if i >= 1:
            p = (i - 1) % 2
            ga[p].wait()
            gb[p].wait()
            _add_pairs(p)
            wh[p] = pltpu.async_copy(
                a[p], out_hbm.at[pl.ds(base + (i - 1) * _CCH, _CCH)], ws[p])
    lp = (n_chunks - 1) % 2
    ga[lp].wait()
    gb[lp].wait()
    _add_pairs(lp)
    wh[lp] = pltpu.async_copy(
        a[lp], out_hbm.at[pl.ds(base + (n_chunks - 1) * _CCH, _CCH)], ws[lp])
    for s in range(2):
        if wh[s] is not None:
            wh[s].wait()


def _sc_combine(y, dest_even, dest_odd):
    k = functools.partial(
        pl.kernel,
        mesh=plsc.VectorSubcoreMesh(core_axis_name="c", subcore_axis_name="s"),
        out_type=jax.ShapeDtypeStruct((B, D), jnp.float32),
        scratch_types=[
            pltpu.VMEM((B // _NW,), jnp.int32),
            pltpu.VMEM((B // _NW,), jnp.int32),
            pltpu.VMEM((_CCH, D), jnp.float32),
            pltpu.VMEM((_CCH, D), jnp.float32),
            pltpu.VMEM((_CCH, D), jnp.float32),
            pltpu.VMEM((_CCH, D), jnp.float32),
            pltpu.SemaphoreType.DMA,
            pltpu.SemaphoreType.DMA,
            pltpu.SemaphoreType.DMA,
            pltpu.SemaphoreType.DMA,
            pltpu.SemaphoreType.DMA,
            pltpu.SemaphoreType.DMA,
        ],
    )(_sc_combine_body)
    return k(y, dest_even, dest_odd)


def kernel(x, Wg, bg, W1, b1, W2, b2):
    idx, wts, dest, pend = _routing(x, Wg, bg)
    dest_even, dest_odd, tok_padded, wt_padded, tile_expert = _metadata(
        idx, wts, dest, pend)
    xg = _sc_gather(x, tok_padded)
    y = _grouped_gemm(xg, W1, b1, W2, b2, wt_padded, tile_expert)
    return _sc_combine(y, dest_even, dest_odd)


# speedup vs baseline: 1.0753x; 1.0064x over previous
import functools
import jax
import jax.numpy as jnp
from jax import lax
from jax.experimental import pallas as pl
from jax.experimental.pallas import tpu as pltpu
from jax.experimental.pallas import tpu_sc as plsc

B, D, E, K = 4096, 768, 8, 2
P = B * K
TR = 256
NP = P + E * TR
NT = NP // TR

_NC, _NS = 2, 16
_NW = _NC * _NS
_GCH = 32
_CCH = 32


def _routing_kernel(x_ref, wg_ref, bg_ref, idx_ref, wts_ref, dest_ref, pend_ref):
    x = x_ref[...]
    glog = jnp.dot(x, wg_ref[...], preferred_element_type=jnp.float32) + bg_ref[...]
    ii = jax.lax.broadcasted_iota(jnp.int32, glog.shape, 1)
    ne = glog.shape[1]
    m1 = jnp.max(glog, axis=1, keepdims=True)
    i1 = jnp.min(jnp.where(glog >= m1, ii, ne), axis=1, keepdims=True)
    neg = jnp.finfo(jnp.float32).min
    g2 = jnp.where(ii == i1, neg, glog)
    m2 = jnp.max(g2, axis=1, keepdims=True)
    i2 = jnp.min(jnp.where(g2 >= m2, ii, ne), axis=1, keepdims=True)
    p2 = jnp.exp(m2 - m1)
    denom = 1.0 + p2
    idx_ref[...] = jnp.concatenate([i1, i2], axis=1)
    wts_ref[...] = jnp.concatenate([1.0 / denom, p2 / denom], axis=1)

    one = ((ii == i1) | (ii == i2)).astype(jnp.int32)
    inc = one
    sh = 1
    while sh < one.shape[0]:
        z = jnp.zeros((sh, one.shape[1]), jnp.int32)
        inc = inc + jnp.concatenate([z, inc[:-sh, :]], axis=0)
        sh *= 2
    excl = inc - one
    counts = inc[-1:, :]
    padded = ((counts + TR - 1) // TR) * TR
    pinc = padded
    sh = 1
    while sh < ne:
        z = jnp.zeros((1, sh), jnp.int32)
        pinc = pinc + jnp.concatenate([z, pinc[:, :-sh]], axis=1)
        sh *= 2
    poff = pinc - padded
    d1 = jnp.sum(jnp.where(ii == i1, poff + excl, 0), axis=1, keepdims=True)
    d2 = jnp.sum(jnp.where(ii == i2, poff + excl, 0), axis=1, keepdims=True)
    dest_ref[...] = jnp.concatenate([d1, d2], axis=1)
    pend_ref[...] = pinc


def _routing(x, Wg, bg):
    return pl.pallas_call(
        _routing_kernel,
        grid=(1,),
        in_specs=[
            pl.BlockSpec((B, D), lambda i: (0, 0)),
            pl.BlockSpec((D, E), lambda i: (0, 0)),
            pl.BlockSpec((1, E), lambda i: (0, 0)),
        ],
        out_specs=[
            pl.BlockSpec((B, K), lambda i: (0, 0)),
            pl.BlockSpec((B, K), lambda i: (0, 0)),
            pl.BlockSpec((B, K), lambda i: (0, 0)),
            pl.BlockSpec((1, E), lambda i: (0, 0)),
        ],
        out_shape=[
            jax.ShapeDtypeStruct((B, K), jnp.int32),
            jax.ShapeDtypeStruct((B, K), jnp.float32),
            jax.ShapeDtypeStruct((B, K), jnp.int32),
            jax.ShapeDtypeStruct((1, E), jnp.int32),
        ],
    )(x, Wg, bg.reshape(1, E))


def _metadata(idx, wts, dest, pend):
    wt = wts.reshape(P)
    tok = jnp.repeat(jnp.arange(B, dtype=jnp.int32), K)
    destf = dest.reshape(P)
    tok_padded = jnp.zeros((NP,), jnp.int32).at[destf].set(tok)
    wt_padded = jnp.zeros((NP,), jnp.float32).at[destf].set(wt)
    t_starts = jnp.arange(NT, dtype=jnp.int32) * TR
    tile_expert = jnp.clip(jnp.sum((t_starts[:, None] >= pend[0][None, :]).astype(jnp.int32),
                                    axis=1), 0, E - 1).astype(jnp.int32)
    return dest[:, 0], dest[:, 1], tok_padded, wt_padded, tile_expert


_NB = 4


def _sc_gather_body(x_hbm, tokp_hbm, xg_hbm, idx_v, *bufs_and_sems):
    bufs = bufs_and_sems[:_NB]
    gsems = bufs_and_sems[_NB:2 * _NB]
    wsems = bufs_and_sems[2 * _NB:3 * _NB]
    wid = lax.axis_index("s") * _NC + lax.axis_index("c")
    rows_per_w = NP // _NW
    n_chunks = rows_per_w // _GCH
    base = wid * rows_per_w
    pltpu.sync_copy(tokp_hbm.at[pl.ds(base, rows_per_w)], idx_v)
    gh = [None] * _NB
    wh = [None] * _NB
    for i in range(n_chunks):
        b = i % _NB
        if wh[b] is not None:
            wh[b].wait()
        gh[b] = pltpu.async_copy(
            x_hbm.at[idx_v.at[pl.ds(i * _GCH, _GCH)]], bufs[b], gsems[b])
        if i >= _NB - 1:
            pb = (i - (_NB - 1)) % _NB
            gh[pb].wait()
            wh[pb] = pltpu.async_copy(
                bufs[pb], xg_hbm.at[pl.ds(base + (i - (_NB - 1)) * _GCH, _GCH)],
                wsems[pb])
    for i in range(max(0, n_chunks - (_NB - 1)), n_chunks):
        b = i % _NB
        gh[b].wait()
        wh[b] = pltpu.async_copy(
            bufs[b], xg_hbm.at[pl.ds(base + i * _GCH, _GCH)], wsems[b])
    for b in range(_NB):
        if wh[b] is not None:
            wh[b].wait()


def _sc_gather(x, tok_padded):
    k = functools.partial(
        pl.kernel,
        mesh=plsc.VectorSubcoreMesh(core_axis_name="c", subcore_axis_name="s"),
        out_type=jax.ShapeDtypeStruct((NP, D), jnp.float32),
        scratch_types=(
            [pltpu.VMEM((NP // _NW,), jnp.int32)]
            + [pltpu.VMEM((_GCH, D), jnp.float32)] * _NB
            + [pltpu.SemaphoreType.DMA] * (2 * _NB)
        ),
    )(_sc_gather_body)
    return k(x, tok_padded)


def _gemm_kernel(te_ref, xg_ref, w1_ref, b1_ref, w2_ref, b2_ref, wt_ref, y_ref):
    xg = xg_ref[...]
    h = jnp.maximum(
        jnp.dot(xg, w1_ref[0], preferred_element_type=jnp.float32) + b1_ref[0], 0.0)
    y = jnp.dot(h, w2_ref[0], preferred_element_type=jnp.float32) + b2_ref[0]
    y_ref[...] = y * wt_ref[...]


def _grouped_gemm(xg, W1, b1, W2, b2, wt_padded, tile_expert):
    grid_spec = pltpu.PrefetchScalarGridSpec(
        num_scalar_prefetch=1,
        grid=(NT,),
        in_specs=[
            pl.BlockSpec((TR, D), lambda t, te: (t, 0)),
            pl.BlockSpec((1, D, D), lambda t, te: (te[t], 0, 0)),
            pl.BlockSpec((1, 1, D), lambda t, te: (te[t], 0, 0)),
            pl.BlockSpec((1, D, D), lambda t, te: (te[t], 0, 0)),
            pl.BlockSpec((1, 1, D), lambda t, te: (te[t], 0, 0)),
            pl.BlockSpec((TR, 1), lambda t, te: (t, 0)),
        ],
        out_specs=pl.BlockSpec((TR, D), lambda t, te: (t, 0)),
    )
    return pl.pallas_call(
        _gemm_kernel,
        grid_spec=grid_spec,
        out_shape=jax.ShapeDtypeStruct((NP, D), jnp.float32),
    )(tile_expert, xg, W1, b1.reshape(E, 1, D), W2, b2.reshape(E, 1, D),
      wt_padded.reshape(NP, 1))


def _sc_combine_body(y_hbm, de_hbm, do_hbm, out_hbm, de_v, do_v,
                     a0, a1, b0, b1, sa0, sa1, sb0, sb1, w0, w1):
    wid = lax.axis_index("s") * _NC + lax.axis_index("c")
    tpw = B // _NW
    n_chunks = tpw // _CCH
    base = wid * tpw
    a = (a0, a1)
    b = (b0, b1)
    gsa = (sa0, sa1)
    gsb = (sb0, sb1)
    ws = (w0, w1)
    pltpu.sync_copy(de_hbm.at[pl.ds(base, tpw)], de_v)
    pltpu.sync_copy(do_hbm.at[pl.ds(base, tpw)], do_v)
    ga = [None, None]
    gb = [None, None]
    wh = [None, None]

    def _add_pairs(s):
        def body(t, carry):
            for j in range(D // 16):
                sl = pl.ds(j * 16, 16)
                a[s][t, sl] = a[s][t, sl] + b[s][t, sl]
            return carry
        lax.fori_loop(0, _CCH, body, 0)

    for i in range(n_chunks):
        s = i % 2
        if wh[s] is not None:
            wh[s].wait()
        ga[s] = pltpu.async_copy(
            y_hbm.at[de_v.at[pl.ds(i * _CCH, _CCH)]], a[s], gsa[s])
        gb[s] = pltpu.async_copy(
            y_hbm.at[do_v.at[pl.ds(i * _CCH, _CCH)]], b[s], gsb[s])
        if i >= 1:
            p = (i - 1) % 2
            ga[p].wait()
            gb[p].wait()
            _add_pairs(p)
            wh[p] = pltpu.async_copy(
                a[p], out_hbm.at[pl.ds(base + (i - 1) * _CCH, _CCH)], ws[p])
    lp = (n_chunks - 1) % 2
    ga[lp].wait()
    gb[lp].wait()
    _add_pairs(lp)
    wh[lp] = pltpu.async_copy(
        a[lp], out_hbm.at[pl.ds(base + (n_chunks - 1) * _CCH, _CCH)], ws[lp])
    for s in range(2):
        if wh[s] is not None:
            wh[s].wait()


def _sc_combine(y, dest_even, dest_odd):
    k = functools.partial(
        pl.kernel,
        mesh=plsc.VectorSubcoreMesh(core_axis_name="c", subcore_axis_name="s"),
        out_type=jax.ShapeDtypeStruct((B, D), jnp.float32),
        scratch_types=[
            pltpu.VMEM((B // _NW,), jnp.int32),
            pltpu.VMEM((B // _NW,), jnp.int32),
            pltpu.VMEM((_CCH, D), jnp.float32),
            pltpu.VMEM((_CCH, D), jnp.float32),
            pltpu.VMEM((_CCH, D), jnp.float32),
            pltpu.VMEM((_CCH, D), jnp.float32),
            pltpu.SemaphoreType.DMA,
            pltpu.SemaphoreType.DMA,
            pltpu.SemaphoreType.DMA,
            pltpu.SemaphoreType.DMA,
            pltpu.SemaphoreType.DMA,
            pltpu.SemaphoreType.DMA,
        ],
    )(_sc_combine_body)
    return k(y, dest_even, dest_odd)


def kernel(x, Wg, bg, W1, b1, W2, b2):
    idx, wts, dest, pend = _routing(x, Wg, bg)
    dest_even, dest_odd, tok_padded, wt_padded, tile_expert = _metadata(
        idx, wts, dest, pend)
    xg = _sc_gather(x, tok_padded)
    y = _grouped_gemm(xg, W1, b1, W2, b2, wt_padded, tile_expert)
    return _sc_combine(y, dest_even, dest_odd)


# speedup vs baseline: 2.6400x; 2.4551x over previous
import jax
import jax.numpy as jnp
from jax.experimental import pallas as pl
from jax.experimental.pallas import tpu as pltpu

_TB = 2048


def _router_kernel(x_ref, wg_ref, bg_ref, wmat_ref):
    x = x_ref[...]
    glog = jnp.dot(x, wg_ref[...], preferred_element_type=jnp.float32) + bg_ref[...]
    ii = jax.lax.broadcasted_iota(jnp.int32, glog.shape, 1)
    ne = glog.shape[1]
    m1 = jnp.max(glog, axis=1, keepdims=True)
    i1 = jnp.min(jnp.where(glog >= m1, ii, ne), axis=1, keepdims=True)
    neg = jnp.finfo(jnp.float32).min
    g2 = jnp.where(ii == i1, neg, glog)
    m2 = jnp.max(g2, axis=1, keepdims=True)
    i2 = jnp.min(jnp.where(g2 >= m2, ii, ne), axis=1, keepdims=True)
    p2 = jnp.exp(m2 - m1)
    denom = 1.0 + p2
    wmat_ref[...] = jnp.where(ii == i1, 1.0 / denom,
                              jnp.where(ii == i2, p2 / denom, 0.0))


def _expert_kernel(x_ref, wmat_ref, w1_ref, b1_ref, w2_ref, b2_ref, out_ref):
    e = pl.program_id(1)
    x = x_ref[...]
    ii = jax.lax.broadcasted_iota(jnp.int32, wmat_ref.shape, 1)
    we = jnp.sum(jnp.where(ii == e, wmat_ref[...], 0.0), axis=1, keepdims=True)
    h = jnp.maximum(
        jnp.dot(x, w1_ref[0], preferred_element_type=jnp.float32) + b1_ref[0], 0.0)
    y = jnp.dot(h, w2_ref[0], preferred_element_type=jnp.float32) + b2_ref[0]
    contrib = we * y

    @pl.when(e == 0)
    def _init():
        out_ref[...] = contrib

    @pl.when(e != 0)
    def _acc():
        out_ref[...] += contrib


def kernel(x, Wg, bg, W1, b1, W2, b2):
    B, D = x.shape
    E = Wg.shape[1]
    wmat = pl.pallas_call(
        _router_kernel,
        grid=(1,),
        in_specs=[
            pl.BlockSpec((B, D), lambda i: (0, 0)),
            pl.BlockSpec((D, E), lambda i: (0, 0)),
            pl.BlockSpec((1, E), lambda i: (0, 0)),
        ],
        out_specs=pl.BlockSpec((B, E), lambda i: (0, 0)),
        out_shape=jax.ShapeDtypeStruct((B, E), jnp.float32),
    )(x, Wg, bg.reshape(1, E))

    nb = B // _TB
    out = pl.pallas_call(
        _expert_kernel,
        grid=(nb, E),
        in_specs=[
            pl.BlockSpec((_TB, D), lambda i, e: (i, 0)),
            pl.BlockSpec((_TB, E), lambda i, e: (i, 0)),
            pl.BlockSpec((1, D, D), lambda i, e: (e, 0, 0)),
            pl.BlockSpec((1, 1, D), lambda i, e: (e, 0, 0)),
            pl.BlockSpec((1, D, D), lambda i, e: (e, 0, 0)),
            pl.BlockSpec((1, 1, D), lambda i, e: (e, 0, 0)),
        ],
        out_specs=pl.BlockSpec((_TB, D), lambda i, e: (i, 0)),
        out_shape=jax.ShapeDtypeStruct((B, D), jnp.float32),
        compiler_params=pltpu.CompilerParams(
            dimension_semantics=("parallel", "arbitrary")),
    )(x, wmat, W1, b1.reshape(E, 1, D), W2, b2.reshape(E, 1, D))
    return out
